# baseline (speedup 1.0000x reference)
import functools

import jax
import jax.numpy as jnp
from jax import lax
from jax.experimental import pallas as pl
from jax.experimental.pallas import tpu as pltpu
from jax.experimental.pallas import tpu_sc as plsc

N_FIELDS = 26
VOCAB = 100000
EMBED_DIM = 32
BATCH = 16384

NC, NS, LANES = 2, 16, 16
NW = NC * NS
BPW = BATCH // NW
CB = 4
NCHUNK = BPW // CB
CHUNK_IDX = CB * N_FIELDS
NBUF = 4


def _make_sc_kernel():
    mesh = plsc.VectorSubcoreMesh(core_axis_name="c", subcore_axis_name="s")

    @functools.partial(
        pl.kernel,
        mesh=mesh,
        out_type=jax.ShapeDtypeStruct((BATCH, EMBED_DIM), jnp.float32),
        compiler_params=pltpu.CompilerParams(use_tc_tiling_on_sc=False),
        scratch_types=(
            [pltpu.VMEM((NCHUNK, CHUNK_IDX), jnp.int32),
             pltpu.VMEM((BPW, EMBED_DIM), jnp.float32)]
            + [pltpu.VMEM((CHUNK_IDX, EMBED_DIM), jnp.float32)
               for _ in range(NBUF)]
            + [pltpu.SemaphoreType.DMA for _ in range(NBUF)]
        ),
    )
    def emb_kernel(table_hbm, idx_hbm, out_hbm, idx_v, out_v, *bufs_sems):
        bufs = bufs_sems[:NBUF]
        sems = bufs_sems[NBUF:]
        wid = lax.axis_index("s") * NC + lax.axis_index("c")

        pltpu.sync_copy(idx_hbm.at[wid], idx_v)

        def start(c, b):
            @pl.when(c < NCHUNK)
            def _():
                pltpu.async_copy(table_hbm.at[idx_v.at[c]], bufs[b], sems[b])

        def wait(b):
            pltpu.make_async_copy(
                table_hbm.at[pl.ds(0, CHUNK_IDX)], bufs[b], sems[b]
            ).wait()

        for b in range(NBUF):
            start(b, b)

        def body(g, carry):
            for b in range(NBUF):
                c = g * NBUF + b
                wait(b)
                buf = bufs[b]
                for j in range(CB):
                    r0 = j * N_FIELDS
                    acc0 = buf[r0, pl.ds(0, LANES)]
                    acc1 = buf[r0, pl.ds(LANES, LANES)]
                    for f in range(1, N_FIELDS):
                        acc0 = acc0 + buf[r0 + f, pl.ds(0, LANES)]
                        acc1 = acc1 + buf[r0 + f, pl.ds(LANES, LANES)]
                    out_r = c * CB + j
                    out_v[out_r, pl.ds(0, LANES)] = acc0
                    out_v[out_r, pl.ds(LANES, LANES)] = acc1
                start(c + NBUF, b)
            return carry

        lax.fori_loop(0, NCHUNK // NBUF, body, 0)

        pltpu.sync_copy(out_v, out_hbm.at[pl.ds(wid * BPW, BPW)])

    return emb_kernel


_emb_kernel = _make_sc_kernel()


@jax.jit
def kernel(x, tables):
    flat_tables = tables.reshape(N_FIELDS * VOCAB, EMBED_DIM)
    offs = (jnp.arange(N_FIELDS, dtype=jnp.int32) * VOCAB)[None, :]
    flat_idx = (x.astype(jnp.int32) + offs).reshape(NW, NCHUNK, CHUNK_IDX)
    return _emb_kernel(flat_tables, flat_idx)
